# Initial kernel scaffold; baseline (speedup 1.0000x reference)
import functools

import jax
import jax.numpy as jnp
from jax.experimental import pallas as pl
from jax.experimental.pallas import tpu as pltpu

_INPUT_DIM = 768
_NUM_EXPERTS = 8
_TOPK = 2
_LOAD_BALANCING_COEF = 0.1

_BLK = 2048


def _router_body(x_ref, wt_ref, p1_ref, p2_ref, i1_ref, i2_ref,
                 agg_ref, cnt_ref, loss_ref):
    step = pl.program_id(0)
    nsteps = pl.num_programs(0)

    @pl.when(step == 0)
    def _init():
        agg_ref[...] = jnp.zeros_like(agg_ref)
        cnt_ref[...] = jnp.zeros_like(cnt_ref)

    x = x_ref[...]
    wt = wt_ref[...]
    logits = jax.lax.dot_general(
        x, wt, (((1,), (0,)), ((), ())),
        preferred_element_type=jnp.float32)

    m = jnp.max(logits, axis=1, keepdims=True)
    e = jnp.exp(logits - m)
    s = jnp.sum(e, axis=1, keepdims=True)
    probs = e / s

    col = jax.lax.broadcasted_iota(jnp.int32, probs.shape, 1)
    m1 = jnp.max(probs, axis=1, keepdims=True)
    i1 = jnp.min(jnp.where(probs == m1, col, _NUM_EXPERTS), axis=1)
    oh1 = col == i1[:, None]
    pm = jnp.where(oh1, -1.0, probs)
    m2 = jnp.max(pm, axis=1, keepdims=True)
    i2 = jnp.min(jnp.where(pm == m2, col, _NUM_EXPERTS), axis=1)
    oh2 = col == i2[:, None]

    p1_ref[...] = m1[:, 0]
    p2_ref[...] = m2[:, 0]
    i1_ref[...] = i1
    i2_ref[...] = i2

    agg_ref[...] += jnp.sum(probs, axis=0)[None, :]
    cnt_ref[...] += jnp.sum(oh1.astype(jnp.float32) + oh2.astype(jnp.float32),
                            axis=0)[None, :]

    @pl.when(step == nsteps - 1)
    def _final():
        num_tokens = nsteps * _BLK
        scale = (_NUM_EXPERTS * _LOAD_BALANCING_COEF
                 / (num_tokens * num_tokens * _TOPK))
        loss_ref[0, 0] = jnp.sum(agg_ref[...] * cnt_ref[...]) * scale


@functools.partial(jax.jit, static_argnames=("interpret",))
def _router(x, wt, interpret=False):
    n = x.shape[0]
    grid = (n // _BLK,)
    out_shapes = (
        jax.ShapeDtypeStruct((n,), jnp.float32),
        jax.ShapeDtypeStruct((n,), jnp.float32),
        jax.ShapeDtypeStruct((n,), jnp.int32),
        jax.ShapeDtypeStruct((n,), jnp.int32),
        jax.ShapeDtypeStruct((1, _NUM_EXPERTS), jnp.float32),
        jax.ShapeDtypeStruct((1, _NUM_EXPERTS), jnp.float32),
        jax.ShapeDtypeStruct((1, 1), jnp.float32),
    )
    vec_spec = pl.BlockSpec((_BLK,), lambda i: (i,))
    acc_spec = pl.BlockSpec((1, _NUM_EXPERTS), lambda i: (0, 0))
    return pl.pallas_call(
        _router_body,
        grid=grid,
        in_specs=[
            pl.BlockSpec((_BLK, _INPUT_DIM), lambda i: (i, 0)),
            pl.BlockSpec((_INPUT_DIM, _NUM_EXPERTS), lambda i: (0, 0)),
        ],
        out_specs=(
            vec_spec, vec_spec, vec_spec, vec_spec,
            acc_spec, acc_spec,
            pl.BlockSpec((1, 1), lambda i: (0, 0)),
        ),
        out_shape=out_shapes,
        compiler_params=pltpu.CompilerParams(
            dimension_semantics=("arbitrary",)),
        interpret=interpret,
    )(x, wt)


def kernel(input, W):
    x = input.reshape(-1, _INPUT_DIM)
    p1, p2, i1, i2, _agg, _cnt, loss = _router(x, W.T)
    top_probs = jnp.stack([p1, p2], axis=1)
    top_indices = jnp.stack([i1, i2], axis=1)
    return top_probs, top_indices, loss[0, 0]


# speedup vs baseline: 1.0733x; 1.0733x over previous
import functools

import jax
import jax.numpy as jnp
from jax.experimental import pallas as pl
from jax.experimental.pallas import tpu as pltpu

_INPUT_DIM = 768
_NUM_EXPERTS = 8
_TOPK = 2
_LOAD_BALANCING_COEF = 0.1

_BLK = 2048


def _router_body(x_ref, wt_ref, p1_ref, p2_ref, i1_ref, i2_ref,
                 agg_ref, cnt_ref, loss_ref):
    step = pl.program_id(0)
    nsteps = pl.num_programs(0)

    @pl.when(step == 0)
    def _init():
        agg_ref[...] = jnp.zeros_like(agg_ref)
        cnt_ref[...] = jnp.zeros_like(cnt_ref)

    x = x_ref[...]
    wt = wt_ref[...]
    logits = jax.lax.dot_general(
        x, wt, (((1,), (0,)), ((), ())),
        preferred_element_type=jnp.float32)

    m = jnp.max(logits, axis=1, keepdims=True)
    e = jnp.exp(logits - m)
    s = jnp.sum(e, axis=1, keepdims=True)
    probs = e / s

    col = jax.lax.broadcasted_iota(jnp.int32, probs.shape, 1)
    m1 = jnp.max(probs, axis=1, keepdims=True)
    i1 = jnp.min(jnp.where(probs == m1, col, _NUM_EXPERTS), axis=1)
    oh1 = col == i1[:, None]
    pm = jnp.where(oh1, -1.0, probs)
    m2 = jnp.max(pm, axis=1, keepdims=True)
    i2 = jnp.min(jnp.where(pm == m2, col, _NUM_EXPERTS), axis=1)
    oh2 = col == i2[:, None]

    p1_ref[...] = m1[:, 0]
    p2_ref[...] = m2[:, 0]
    i1_ref[...] = i1
    i2_ref[...] = i2

    agg_ref[...] += jnp.sum(probs, axis=0)[None, :]
    cnt_ref[...] += jnp.sum(oh1.astype(jnp.float32) + oh2.astype(jnp.float32),
                            axis=0)[None, :]

    @pl.when(step == nsteps - 1)
    def _final():
        num_tokens = nsteps * _BLK
        scale = (_NUM_EXPERTS * _LOAD_BALANCING_COEF
                 / (num_tokens * num_tokens * _TOPK))
        loss_ref[...] = (jnp.sum(agg_ref[...] * cnt_ref[...])
                         * scale).reshape(1, 1)


@functools.partial(jax.jit, static_argnames=("interpret",))
def _router(x, wt, interpret=False):
    n = x.shape[0]
    grid = (n // _BLK,)
    out_shapes = (
        jax.ShapeDtypeStruct((n,), jnp.float32),
        jax.ShapeDtypeStruct((n,), jnp.float32),
        jax.ShapeDtypeStruct((n,), jnp.int32),
        jax.ShapeDtypeStruct((n,), jnp.int32),
        jax.ShapeDtypeStruct((1, _NUM_EXPERTS), jnp.float32),
        jax.ShapeDtypeStruct((1, _NUM_EXPERTS), jnp.float32),
        jax.ShapeDtypeStruct((1, 1), jnp.float32),
    )
    vec_spec = pl.BlockSpec((_BLK,), lambda i: (i,))
    acc_spec = pl.BlockSpec((1, _NUM_EXPERTS), lambda i: (0, 0))
    return pl.pallas_call(
        _router_body,
        grid=grid,
        in_specs=[
            pl.BlockSpec((_BLK, _INPUT_DIM), lambda i: (i, 0)),
            pl.BlockSpec((_INPUT_DIM, _NUM_EXPERTS), lambda i: (0, 0)),
        ],
        out_specs=(
            vec_spec, vec_spec, vec_spec, vec_spec,
            acc_spec, acc_spec,
            pl.BlockSpec((1, 1), lambda i: (0, 0)),
        ),
        out_shape=out_shapes,
        compiler_params=pltpu.CompilerParams(
            dimension_semantics=("arbitrary",)),
        interpret=interpret,
    )(x, wt)


def kernel(input, W):
    x = input.reshape(-1, _INPUT_DIM)
    p1, p2, i1, i2, _agg, _cnt, loss = _router(x, W.T)
    top_probs = jnp.stack([p1, p2], axis=1)
    top_indices = jnp.stack([i1, i2], axis=1)
    return top_probs, top_indices, loss[0, 0]


# speedup vs baseline: 1.0790x; 1.0053x over previous
import functools

import jax
import jax.numpy as jnp
from jax.experimental import pallas as pl
from jax.experimental.pallas import tpu as pltpu

_INPUT_DIM = 768
_NUM_EXPERTS = 8
_TOPK = 2
_LOAD_BALANCING_COEF = 0.1

_BLK = 4096


def _router_body(x_ref, wt_ref, p1_ref, p2_ref, i1_ref, i2_ref,
                 agg_ref, cnt_ref, loss_ref):
    step = pl.program_id(0)
    nsteps = pl.num_programs(0)

    @pl.when(step == 0)
    def _init():
        agg_ref[...] = jnp.zeros_like(agg_ref)
        cnt_ref[...] = jnp.zeros_like(cnt_ref)

    x = x_ref[...]
    wt = wt_ref[...]
    logits = jax.lax.dot_general(
        x, wt, (((1,), (0,)), ((), ())),
        preferred_element_type=jnp.float32)

    m = jnp.max(logits, axis=1, keepdims=True)
    e = jnp.exp(logits - m)
    s = jnp.sum(e, axis=1, keepdims=True)
    probs = e / s

    col = jax.lax.broadcasted_iota(jnp.int32, probs.shape, 1)
    m1 = jnp.max(probs, axis=1, keepdims=True)
    i1 = jnp.min(jnp.where(probs == m1, col, _NUM_EXPERTS), axis=1)
    oh1 = col == i1[:, None]
    pm = jnp.where(oh1, -1.0, probs)
    m2 = jnp.max(pm, axis=1, keepdims=True)
    i2 = jnp.min(jnp.where(pm == m2, col, _NUM_EXPERTS), axis=1)
    oh2 = col == i2[:, None]

    p1_ref[...] = m1[:, 0]
    p2_ref[...] = m2[:, 0]
    i1_ref[...] = i1
    i2_ref[...] = i2

    agg_ref[...] += jnp.sum(probs, axis=0)[None, :]
    cnt_ref[...] += jnp.sum(oh1.astype(jnp.float32) + oh2.astype(jnp.float32),
                            axis=0)[None, :]

    @pl.when(step == nsteps - 1)
    def _final():
        num_tokens = nsteps * _BLK
        scale = (_NUM_EXPERTS * _LOAD_BALANCING_COEF
                 / (num_tokens * num_tokens * _TOPK))
        loss_ref[...] = (jnp.sum(agg_ref[...] * cnt_ref[...])
                         * scale).reshape(1, 1)


@functools.partial(jax.jit, static_argnames=("interpret",))
def _router(x, wt, interpret=False):
    n = x.shape[0]
    grid = (n // _BLK,)
    out_shapes = (
        jax.ShapeDtypeStruct((n,), jnp.float32),
        jax.ShapeDtypeStruct((n,), jnp.float32),
        jax.ShapeDtypeStruct((n,), jnp.int32),
        jax.ShapeDtypeStruct((n,), jnp.int32),
        jax.ShapeDtypeStruct((1, _NUM_EXPERTS), jnp.float32),
        jax.ShapeDtypeStruct((1, _NUM_EXPERTS), jnp.float32),
        jax.ShapeDtypeStruct((1, 1), jnp.float32),
    )
    vec_spec = pl.BlockSpec((_BLK,), lambda i: (i,))
    acc_spec = pl.BlockSpec((1, _NUM_EXPERTS), lambda i: (0, 0))
    return pl.pallas_call(
        _router_body,
        grid=grid,
        in_specs=[
            pl.BlockSpec((_BLK, _INPUT_DIM), lambda i: (i, 0)),
            pl.BlockSpec((_INPUT_DIM, _NUM_EXPERTS), lambda i: (0, 0)),
        ],
        out_specs=(
            vec_spec, vec_spec, vec_spec, vec_spec,
            acc_spec, acc_spec,
            pl.BlockSpec((1, 1), lambda i: (0, 0)),
        ),
        out_shape=out_shapes,
        compiler_params=pltpu.CompilerParams(
            dimension_semantics=("arbitrary",)),
        interpret=interpret,
    )(x, wt)


def kernel(input, W):
    x = input.reshape(-1, _INPUT_DIM)
    p1, p2, i1, i2, _agg, _cnt, loss = _router(x, W.T)
    top_probs = jnp.stack([p1, p2], axis=1)
    top_indices = jnp.stack([i1, i2], axis=1)
    return top_probs, top_indices, loss[0, 0]
